# Initial kernel scaffold; baseline (speedup 1.0000x reference)
import jax
import jax.numpy as jnp
from jax.experimental import pallas as pl


def kernel(x, seg, tok_w, pos_w, seg_w, gamma, beta):
    raise NotImplementedError("write your pallas kernel here")


# speedup vs baseline: 5.1807x; 5.1807x over previous
import functools

import jax
import jax.numpy as jnp
from jax import lax
from jax.experimental import pallas as pl
from jax.experimental.pallas import tpu as pltpu
from jax.experimental.pallas import tpu_sc as plsc

VOCAB = 26
NSEG = 15
DM = 512
B = 4096
S = 64
COMBO = 400
NTOK = B * S

NC, NS = 2, 16
NW = NC * NS
TPW = NTOK // NW
G = 64
NG = TPW // G


def _table_body(tok_ref, seg_ref, pos_ref, g_ref, b_ref, out_ref):
    r_v = lax.broadcasted_iota(jnp.int32, (COMBO, 32), 0)
    c_v = lax.broadcasted_iota(jnp.int32, (COMBO, 32), 1)
    ohv = (r_v // NSEG == c_v).astype(jnp.float32)
    r_g = lax.broadcasted_iota(jnp.int32, (COMBO, 16), 0)
    c_g = lax.broadcasted_iota(jnp.int32, (COMBO, 16), 1)
    ohg = (r_g % NSEG == c_g).astype(jnp.float32)
    emb = (
        lax.dot(ohv, tok_ref[...], precision=lax.Precision.HIGHEST)
        + lax.dot(ohg, seg_ref[...], precision=lax.Precision.HIGHEST)
        + pos_ref[pl.ds(pl.program_id(0), 1), :]
    )
    mean = jnp.mean(emb, axis=1, keepdims=True)
    var = jnp.mean((emb - mean) ** 2, axis=1, keepdims=True)
    normed = (emb - mean) / jnp.sqrt(var + 1e-5)
    out_ref[...] = normed * g_ref[...] + b_ref[...]


_table_call = pl.pallas_call(
    _table_body,
    grid=(S,),
    in_specs=[
        pl.BlockSpec((32, DM), lambda s: (0, 0)),
        pl.BlockSpec((16, DM), lambda s: (0, 0)),
        pl.BlockSpec((S, DM), lambda s: (0, 0)),
        pl.BlockSpec((1, DM), lambda s: (0, 0)),
        pl.BlockSpec((1, DM), lambda s: (0, 0)),
    ],
    out_specs=pl.BlockSpec((COMBO, DM), lambda s: (s, 0)),
    out_shape=jax.ShapeDtypeStruct((S * COMBO, DM), jnp.float32),
)


def _cidx_body(x_ref, seg_ref, out_ref):
    pos = lax.broadcasted_iota(jnp.int32, (B, S), 1)
    out_ref[...] = COMBO * pos + NSEG * x_ref[...] + seg_ref[...]


_cidx_call = pl.pallas_call(
    _cidx_body,
    out_shape=jax.ShapeDtypeStruct((B, S), jnp.int32),
)


@functools.cache
def _sc_gather_call():
    mesh = plsc.VectorSubcoreMesh(
        core_axis_name="c", subcore_axis_name="s", num_cores=NC, num_subcores=NS
    )

    @functools.partial(
        pl.kernel,
        out_type=jax.ShapeDtypeStruct((NTOK, DM), jnp.float32),
        mesh=mesh,
        scratch_types=[
            pltpu.VMEM((NG, G), jnp.int32),
            pltpu.VMEM((G, DM), jnp.float32),
            pltpu.SemaphoreType.DMA,
        ],
    )
    def _sc_gather(table_hbm, cidx_hbm, out_hbm, idx_v, rows_v, sem):
        wid = lax.axis_index("s") * NC + lax.axis_index("c")
        pltpu.sync_copy(cidx_hbm.at[wid], idx_v)
        base = wid * TPW

        def step(j, carry):
            pltpu.async_copy(table_hbm.at[idx_v.at[j]], rows_v, sem).wait()
            off = pl.multiple_of(base + j * G, G)
            pltpu.sync_copy(rows_v, out_hbm.at[pl.ds(off, G)])
            return carry

        lax.fori_loop(0, NG, step, 0)

    return _sc_gather


def kernel(x, seg, tok_w, pos_w, seg_w, gamma, beta):
    tok_pad = jnp.zeros((32, DM), jnp.float32).at[:VOCAB].set(tok_w)
    seg_pad = jnp.zeros((16, DM), jnp.float32).at[:NSEG].set(seg_w)
    table = _table_call(
        tok_pad, seg_pad, pos_w[:S], gamma.reshape(1, DM), beta.reshape(1, DM)
    )
    cidx = _cidx_call(x.astype(jnp.int32), seg.astype(jnp.int32))
    out = _sc_gather_call()(table, cidx.reshape(NW, NG, G))
    return out.reshape(B, S, DM)


# speedup vs baseline: 7.0326x; 1.3575x over previous
import functools

import jax
import jax.numpy as jnp
from jax import lax
from jax.experimental import pallas as pl
from jax.experimental.pallas import tpu as pltpu
from jax.experimental.pallas import tpu_sc as plsc

VOCAB = 26
NSEG = 15
DM = 512
B = 4096
S = 64
COMBO = 400
NTOK = B * S

NC, NS = 2, 16
NW = NC * NS
TPW = NTOK // NW
G = 64
NG = TPW // G


def _table_body(tok_ref, seg_ref, pos_ref, g_ref, b_ref, out_ref, tokseg_ref):
    @pl.when(pl.program_id(0) == 0)
    def _():
        r_v = lax.broadcasted_iota(jnp.int32, (COMBO, 32), 0)
        c_v = lax.broadcasted_iota(jnp.int32, (COMBO, 32), 1)
        ohv = (r_v // NSEG == c_v).astype(jnp.float32)
        r_g = lax.broadcasted_iota(jnp.int32, (COMBO, 16), 0)
        c_g = lax.broadcasted_iota(jnp.int32, (COMBO, 16), 1)
        ohg = (r_g % NSEG == c_g).astype(jnp.float32)
        tokseg_ref[...] = lax.dot(
            ohv, tok_ref[...], precision=lax.Precision.HIGHEST
        ) + lax.dot(ohg, seg_ref[...], precision=lax.Precision.HIGHEST)

    emb = tokseg_ref[...] + pos_ref[pl.ds(pl.program_id(0), 1), :]
    mean = jnp.mean(emb, axis=1, keepdims=True)
    var = jnp.mean((emb - mean) ** 2, axis=1, keepdims=True)
    normed = (emb - mean) / jnp.sqrt(var + 1e-5)
    out_ref[...] = normed * g_ref[...] + b_ref[...]


_table_call = pl.pallas_call(
    _table_body,
    grid=(S,),
    in_specs=[
        pl.BlockSpec((32, DM), lambda s: (0, 0)),
        pl.BlockSpec((16, DM), lambda s: (0, 0)),
        pl.BlockSpec((S, DM), lambda s: (0, 0)),
        pl.BlockSpec((1, DM), lambda s: (0, 0)),
        pl.BlockSpec((1, DM), lambda s: (0, 0)),
    ],
    out_specs=pl.BlockSpec((COMBO, DM), lambda s: (s, 0)),
    out_shape=jax.ShapeDtypeStruct((S * COMBO, DM), jnp.float32),
    scratch_shapes=[pltpu.VMEM((COMBO, DM), jnp.float32)],
)


def _cidx_body(x_ref, seg_ref, out_ref):
    pos = lax.broadcasted_iota(jnp.int32, (B, S), 1)
    out_ref[...] = COMBO * pos + NSEG * x_ref[...] + seg_ref[...]


_cidx_call = pl.pallas_call(
    _cidx_body,
    out_shape=jax.ShapeDtypeStruct((B, S), jnp.int32),
)


@functools.cache
def _sc_gather_call():
    mesh = plsc.VectorSubcoreMesh(
        core_axis_name="c", subcore_axis_name="s", num_cores=NC, num_subcores=NS
    )

    @functools.partial(
        pl.kernel,
        out_type=jax.ShapeDtypeStruct((NTOK, DM), jnp.float32),
        mesh=mesh,
        scratch_types=[
            pltpu.VMEM((NG, G), jnp.int32),
            pltpu.VMEM((G, DM), jnp.float32),
            pltpu.VMEM((G, DM), jnp.float32),
            pltpu.SemaphoreType.DMA,
            pltpu.SemaphoreType.DMA,
        ],
    )
    def _sc_gather(table_hbm, cidx_hbm, out_hbm, idx_v, rows_a, rows_b, sem_a, sem_b):
        wid = lax.axis_index("s") * NC + lax.axis_index("c")
        pltpu.sync_copy(cidx_hbm.at[wid], idx_v)
        base = wid * TPW

        def gather(j, buf, sem):
            pltpu.async_copy(table_hbm.at[idx_v.at[j]], buf, sem)

        def drain(buf, sem):
            pltpu.make_async_copy(table_hbm.at[pl.ds(0, G)], buf, sem).wait()

        def store(j, buf):
            off = pl.multiple_of(base + j * G, G)
            pltpu.sync_copy(buf, out_hbm.at[pl.ds(off, G)])

        gather(0, rows_a, sem_a)

        def pair(i, carry):
            j = 2 * i
            gather(j + 1, rows_b, sem_b)
            drain(rows_a, sem_a)
            store(j, rows_a)

            @pl.when(j + 2 < NG)
            def _():
                gather(j + 2, rows_a, sem_a)

            drain(rows_b, sem_b)
            store(j + 1, rows_b)
            return carry

        lax.fori_loop(0, NG // 2, pair, 0)

    return _sc_gather


def kernel(x, seg, tok_w, pos_w, seg_w, gamma, beta):
    tok_pad = jnp.zeros((32, DM), jnp.float32).at[:VOCAB].set(tok_w)
    seg_pad = jnp.zeros((16, DM), jnp.float32).at[:NSEG].set(seg_w)
    table = _table_call(
        tok_pad, seg_pad, pos_w[:S], gamma.reshape(1, DM), beta.reshape(1, DM)
    )
    cidx = _cidx_call(x.astype(jnp.int32), seg.astype(jnp.int32))
    out = _sc_gather_call()(table, cidx.reshape(NW, NG, G))
    return out.reshape(B, S, DM)
